# Initial kernel scaffold; baseline (speedup 1.0000x reference)
import jax
import jax.numpy as jnp
from jax.experimental import pallas as pl


def kernel(x, adj, W0, a0, W1, a1, W2, a2, W_out, a_out):
    raise NotImplementedError("write your pallas kernel here")


# speedup vs baseline: 6.4034x; 6.4034x over previous
import functools

import jax
import jax.numpy as jnp
from jax import lax
from jax.experimental import pallas as pl
from jax.experimental.pallas import tpu as pltpu
from jax.experimental.pallas import tpu_sc as plsc

N = 10000
E = 320000
NFEAT = 128
NHID = 64
NCLASS = 64
NHEADS = 3
ALPHA = 0.2

NC = 2
NS = 16
L = 16
CHUNK = 80
CHUNKS_PER_TILE = (E // NC) // (NS * CHUNK)
ROWS_PER_TILE = N // NS

_ELU = lambda v: jnp.where(v > 0, v, jnp.exp(v) - 1.0)


def _tc1_body(x_ref, wcat_ref, acat_a_ref, acat_b_ref,
              h01_ref, h2_ref, scal_a_ref, scal_b_ref):
    h = jnp.dot(x_ref[...], wcat_ref[...], preferred_element_type=jnp.float32)
    h01_ref[...] = h[:, :2 * NHID]
    h2_ref[...] = h[:, 2 * NHID:]
    scal_a_ref[...] = jnp.dot(h, acat_a_ref[...], preferred_element_type=jnp.float32)
    scal_b_ref[...] = jnp.dot(h, acat_b_ref[...], preferred_element_type=jnp.float32)


def _tc1(x, wcat, acat_a, acat_b):
    blk = 1000
    return pl.pallas_call(
        _tc1_body,
        grid=(N // blk,),
        in_specs=[
            pl.BlockSpec((blk, NFEAT), lambda i: (i, 0)),
            pl.BlockSpec((NFEAT, NHEADS * NHID), lambda i: (0, 0)),
            pl.BlockSpec((NHEADS * NHID, 8), lambda i: (0, 0)),
            pl.BlockSpec((NHEADS * NHID, 8), lambda i: (0, 0)),
        ],
        out_specs=[
            pl.BlockSpec((blk, 2 * NHID), lambda i: (i, 0)),
            pl.BlockSpec((blk, NHID), lambda i: (i, 0)),
            pl.BlockSpec((blk, 8), lambda i: (i, 0)),
            pl.BlockSpec((blk, 8), lambda i: (i, 0)),
        ],
        out_shape=[
            jax.ShapeDtypeStruct((N, 2 * NHID), jnp.float32),
            jax.ShapeDtypeStruct((N, NHID), jnp.float32),
            jax.ShapeDtypeStruct((N, 8), jnp.float32),
            jax.ShapeDtypeStruct((N, 8), jnp.float32),
        ],
    )(x, wcat, acat_a, acat_b)


def _tc2_body(hpa_ref, rsa_ref, hpb_ref, rsb_ref, wout_ref, aout_ref,
              h2_ref, scal2_ref):
    hsa = hpa_ref[0] + hpa_ref[1]
    rsa = rsa_ref[0] + rsa_ref[1]
    hsb = hpb_ref[0] + hpb_ref[1]
    rsb = rsb_ref[0] + rsb_ref[1]
    parts = []
    for i in range(2):
        hi = hsa[:, i * NHID:(i + 1) * NHID] / (rsa[:, i:i + 1] + 1e-16)
        parts.append(_ELU(hi))
    parts.append(_ELU(hsb / (rsb[:, 0:1] + 1e-16)))
    hcat = jnp.concatenate(parts, axis=1)
    h2 = jnp.dot(hcat, wout_ref[...], preferred_element_type=jnp.float32)
    h2_ref[...] = h2
    scal2_ref[...] = jnp.dot(h2, aout_ref[...], preferred_element_type=jnp.float32)


def _tc2(hpa, rsa, hpb, rsb, wout, aout):
    blk = 1000
    return pl.pallas_call(
        _tc2_body,
        grid=(N // blk,),
        in_specs=[
            pl.BlockSpec((2, blk, 2 * NHID), lambda i: (0, i, 0)),
            pl.BlockSpec((2, blk, 8), lambda i: (0, i, 0)),
            pl.BlockSpec((2, blk, NHID), lambda i: (0, i, 0)),
            pl.BlockSpec((2, blk, 8), lambda i: (0, i, 0)),
            pl.BlockSpec((NHEADS * NHID, NCLASS), lambda i: (0, 0)),
            pl.BlockSpec((NCLASS, 8), lambda i: (0, 0)),
        ],
        out_specs=[
            pl.BlockSpec((blk, NCLASS), lambda i: (i, 0)),
            pl.BlockSpec((blk, 8), lambda i: (i, 0)),
        ],
        out_shape=[
            jax.ShapeDtypeStruct((N, NCLASS), jnp.float32),
            jax.ShapeDtypeStruct((N, 8), jnp.float32),
        ],
    )(hpa, rsa, hpb, rsb, wout, aout)


def _tc3_body(hp_ref, rs_ref, out_ref):
    hsum = hp_ref[0] + hp_ref[1]
    rsum = rs_ref[0] + rs_ref[1]
    out_ref[...] = _ELU(hsum / (rsum[:, 0:1] + 1e-16))


def _tc3(hp, rs):
    blk = 1000
    return pl.pallas_call(
        _tc3_body,
        grid=(N // blk,),
        in_specs=[
            pl.BlockSpec((2, blk, NCLASS), lambda i: (0, i, 0)),
            pl.BlockSpec((2, blk, 8), lambda i: (0, i, 0)),
        ],
        out_specs=pl.BlockSpec((blk, NCLASS), lambda i: (i, 0)),
        out_shape=jax.ShapeDtypeStruct((N, NCLASS), jnp.float32),
    )(hp, rs)


def _sc_edge_kernel(d_feat, n_heads):
    groups = CHUNK // L
    sub = d_feat // L

    def body(h_hbm, scal_hbm, adj_hbm, hp_hbm, rs_hbm,
             src_idx, dst_idx, ss_buf, sd_buf, h_buf, e_buf, hp_sh, rs_sh, sem):
        c = lax.axis_index("c")
        s = lax.axis_index("s")
        iota = lax.iota(jnp.int32, L)
        zeros = jnp.zeros((L,), jnp.float32)

        def zero_h(t, _):
            row = t // sub
            col = (t % sub) * L
            plsc.store_scatter(h_buf, [jnp.full((L,), row, jnp.int32), col + iota],
                               zeros)
            return 0
        lax.fori_loop(0, CHUNK * sub, zero_h, 0)
        for t in range(CHUNK * 8 // L):
            f = t * L + iota
            plsc.store_scatter(e_buf, [f >> 3, f & 7], zeros)

        base_row = s * ROWS_PER_TILE
        n_full = ROWS_PER_TILE // CHUNK
        rem = ROWS_PER_TILE - n_full * CHUNK
        for k in range(n_full):
            pltpu.sync_copy(h_buf, hp_sh.at[pl.ds(base_row + k * CHUNK, CHUNK)])
            pltpu.sync_copy(e_buf, rs_sh.at[pl.ds(base_row + k * CHUNK, CHUNK)])
        pltpu.sync_copy(h_buf.at[pl.ds(0, rem)],
                        hp_sh.at[pl.ds(base_row + n_full * CHUNK, rem)])
        pltpu.sync_copy(e_buf.at[pl.ds(0, rem)],
                        rs_sh.at[pl.ds(base_row + n_full * CHUNK, rem)])
        plsc.subcore_barrier()

        def chunk_body(g, _):
            base = c * (E // NC) + (g * NS + s) * CHUNK
            pltpu.sync_copy(adj_hbm.at[0, pl.ds(base, CHUNK)], src_idx)
            pltpu.sync_copy(adj_hbm.at[1, pl.ds(base, CHUNK)], dst_idx)
            cp1 = pltpu.async_copy(scal_hbm.at[src_idx], ss_buf, sem)
            cp2 = pltpu.async_copy(scal_hbm.at[dst_idx], sd_buf, sem)
            cp3 = pltpu.async_copy(h_hbm.at[dst_idx], h_buf, sem)
            cp1.wait()
            cp2.wait()
            cp3.wait()

            for i in range(n_heads):
                for t in range(groups):
                    r = t * L + iota
                    zs = plsc.load_gather(ss_buf, [r, jnp.full((L,), i, jnp.int32)])
                    zd = plsc.load_gather(sd_buf, [r, jnp.full((L,), n_heads + i,
                                                               jnp.int32)])
                    z = zs + zd
                    e = jnp.exp(-jnp.maximum(z, ALPHA * z))
                    plsc.store_scatter(e_buf, [r, jnp.full((L,), i, jnp.int32)], e)

            def scale_row(j, _):
                jv = jnp.full((L,), j, jnp.int32)
                for i in range(n_heads):
                    es = plsc.load_gather(e_buf, [jv, jnp.full((L,), i, jnp.int32)])
                    for k in range(NHID // L):
                        col = i * NHID + k * L
                        h_buf[j, pl.ds(col, L)] = h_buf[j, pl.ds(col, L)] * es
                return 0
            lax.fori_loop(0, CHUNK, scale_row, 0)

            pltpu.sync_copy(h_buf, hp_sh.at[src_idx], add=True)
            pltpu.sync_copy(e_buf, rs_sh.at[src_idx], add=True)
            return 0
        lax.fori_loop(0, CHUNKS_PER_TILE, chunk_body, 0)
        plsc.subcore_barrier()

        pltpu.sync_copy(hp_sh.at[pl.ds(base_row, ROWS_PER_TILE)],
                        hp_hbm.at[c, pl.ds(base_row, ROWS_PER_TILE)])
        pltpu.sync_copy(rs_sh.at[pl.ds(base_row, ROWS_PER_TILE)],
                        rs_hbm.at[c, pl.ds(base_row, ROWS_PER_TILE)])

    return pl.kernel(
        body,
        out_type=(
            jax.ShapeDtypeStruct((NC, N, d_feat), jnp.float32),
            jax.ShapeDtypeStruct((NC, N, 8), jnp.float32),
        ),
        mesh=plsc.VectorSubcoreMesh(core_axis_name="c", subcore_axis_name="s"),
        compiler_params=pltpu.CompilerParams(use_tc_tiling_on_sc=False,
                                             needs_layout_passes=False),
        scratch_types=[
            pltpu.VMEM((CHUNK,), jnp.int32),
            pltpu.VMEM((CHUNK,), jnp.int32),
            pltpu.VMEM((CHUNK, 8), jnp.float32),
            pltpu.VMEM((CHUNK, 8), jnp.float32),
            pltpu.VMEM((CHUNK, d_feat), jnp.float32),
            pltpu.VMEM((CHUNK, 8), jnp.float32),
            pltpu.VMEM_SHARED((N, d_feat), jnp.float32),
            pltpu.VMEM_SHARED((N, 8), jnp.float32),
            pltpu.SemaphoreType.DMA,
        ],
    )


@jax.jit
def kernel(x, adj, W0, a0, W1, a1, W2, a2, W_out, a_out):
    din = NHEADS * NHID
    wcat = jnp.concatenate([W0, W1, W2], axis=1)
    acat_a = jnp.zeros((din, 8), jnp.float32)
    for i, a in enumerate((a0, a1)):
        acat_a = acat_a.at[i * NHID:(i + 1) * NHID, i].set(a[0, :NHID])
        acat_a = acat_a.at[i * NHID:(i + 1) * NHID, 2 + i].set(a[0, NHID:])
    acat_b = jnp.zeros((din, 8), jnp.float32)
    acat_b = acat_b.at[2 * NHID:, 0].set(a2[0, :NHID])
    acat_b = acat_b.at[2 * NHID:, 1].set(a2[0, NHID:])
    aout = jnp.zeros((NCLASS, 8), jnp.float32)
    aout = aout.at[:, 0].set(a_out[0, :NCLASS])
    aout = aout.at[:, 1].set(a_out[0, NCLASS:])

    h01, h2h, scal_a, scal_b = _tc1(x, wcat, acat_a, acat_b)
    hpa, rsa = _sc_edge_kernel(2 * NHID, 2)(h01, scal_a, adj)
    hpb, rsb = _sc_edge_kernel(NHID, 1)(h2h, scal_b, adj)
    h2, scal2 = _tc2(hpa, rsa, hpb, rsb, W_out, aout)
    hp2, rs2 = _sc_edge_kernel(NCLASS, 1)(h2, scal2, adj)
    return _tc3(hp2, rs2)


# speedup vs baseline: 13.1563x; 2.0546x over previous
import functools

import jax
import jax.numpy as jnp
from jax import lax
from jax.experimental import pallas as pl
from jax.experimental.pallas import tpu as pltpu
from jax.experimental.pallas import tpu_sc as plsc

N = 10000
E = 320000
NFEAT = 128
NHID = 64
NCLASS = 64
NHEADS = 3
ALPHA = 0.2

NC = 2
NS = 16
L = 16
CHUNK = 80
CHUNKS_PER_TILE = (E // NC) // (NS * CHUNK)
ROWS_PER_TILE = N // NS

_ELU = lambda v: jnp.where(v > 0, v, jnp.exp(v) - 1.0)


def _tc1_body(x_ref, wcat_ref, acat_a_ref, acat_b_ref,
              h01_ref, h2_ref, scal_a_ref, scal_b_ref):
    h = jnp.dot(x_ref[...], wcat_ref[...], preferred_element_type=jnp.float32)
    h01_ref[...] = h[:, :2 * NHID]
    h2_ref[...] = h[:, 2 * NHID:]
    scal_a_ref[...] = jnp.dot(h, acat_a_ref[...], preferred_element_type=jnp.float32)
    scal_b_ref[...] = jnp.dot(h, acat_b_ref[...], preferred_element_type=jnp.float32)


def _tc1(x, wcat, acat_a, acat_b):
    blk = 1000
    return pl.pallas_call(
        _tc1_body,
        grid=(N // blk,),
        in_specs=[
            pl.BlockSpec((blk, NFEAT), lambda i: (i, 0)),
            pl.BlockSpec((NFEAT, NHEADS * NHID), lambda i: (0, 0)),
            pl.BlockSpec((NHEADS * NHID, 8), lambda i: (0, 0)),
            pl.BlockSpec((NHEADS * NHID, 8), lambda i: (0, 0)),
        ],
        out_specs=[
            pl.BlockSpec((blk, 2 * NHID), lambda i: (i, 0)),
            pl.BlockSpec((blk, NHID), lambda i: (i, 0)),
            pl.BlockSpec((blk, 8), lambda i: (i, 0)),
            pl.BlockSpec((blk, 8), lambda i: (i, 0)),
        ],
        out_shape=[
            jax.ShapeDtypeStruct((N, 2 * NHID), jnp.float32),
            jax.ShapeDtypeStruct((N, NHID), jnp.float32),
            jax.ShapeDtypeStruct((N, 8), jnp.float32),
            jax.ShapeDtypeStruct((N, 8), jnp.float32),
        ],
    )(x, wcat, acat_a, acat_b)


def _tc2_body(hpa_ref, rsa_ref, hpb_ref, rsb_ref, wout_ref, aout_ref,
              h2_ref, scal2_ref):
    hsa = hpa_ref[0] + hpa_ref[1]
    rsa = rsa_ref[0] + rsa_ref[1]
    hsb = hpb_ref[0] + hpb_ref[1]
    rsb = rsb_ref[0] + rsb_ref[1]
    parts = []
    for i in range(2):
        hi = hsa[:, i * NHID:(i + 1) * NHID] / (rsa[:, i:i + 1] + 1e-16)
        parts.append(_ELU(hi))
    parts.append(_ELU(hsb / (rsb[:, 0:1] + 1e-16)))
    hcat = jnp.concatenate(parts, axis=1)
    h2 = jnp.dot(hcat, wout_ref[...], preferred_element_type=jnp.float32)
    h2_ref[...] = h2
    scal2_ref[...] = jnp.dot(h2, aout_ref[...], preferred_element_type=jnp.float32)


def _tc2(hpa, rsa, hpb, rsb, wout, aout):
    blk = 1000
    return pl.pallas_call(
        _tc2_body,
        grid=(N // blk,),
        in_specs=[
            pl.BlockSpec((2, blk, 2 * NHID), lambda i: (0, i, 0)),
            pl.BlockSpec((2, blk, 8), lambda i: (0, i, 0)),
            pl.BlockSpec((2, blk, NHID), lambda i: (0, i, 0)),
            pl.BlockSpec((2, blk, 8), lambda i: (0, i, 0)),
            pl.BlockSpec((NHEADS * NHID, NCLASS), lambda i: (0, 0)),
            pl.BlockSpec((NCLASS, 8), lambda i: (0, 0)),
        ],
        out_specs=[
            pl.BlockSpec((blk, NCLASS), lambda i: (i, 0)),
            pl.BlockSpec((blk, 8), lambda i: (i, 0)),
        ],
        out_shape=[
            jax.ShapeDtypeStruct((N, NCLASS), jnp.float32),
            jax.ShapeDtypeStruct((N, 8), jnp.float32),
        ],
    )(hpa, rsa, hpb, rsb, wout, aout)


def _tc3_body(hp_ref, rs_ref, out_ref):
    hsum = hp_ref[0] + hp_ref[1]
    rsum = rs_ref[0] + rs_ref[1]
    out_ref[...] = _ELU(hsum / (rsum[:, 0:1] + 1e-16))


def _tc3(hp, rs):
    blk = 1000
    return pl.pallas_call(
        _tc3_body,
        grid=(N // blk,),
        in_specs=[
            pl.BlockSpec((2, blk, NCLASS), lambda i: (0, i, 0)),
            pl.BlockSpec((2, blk, 8), lambda i: (0, i, 0)),
        ],
        out_specs=pl.BlockSpec((blk, NCLASS), lambda i: (i, 0)),
        out_shape=jax.ShapeDtypeStruct((N, NCLASS), jnp.float32),
    )(hp, rs)


def _sc_edge_kernel(d_feat, n_heads):
    groups = CHUNK // L
    sub = d_feat // L
    n_chunks = CHUNKS_PER_TILE

    def body(h_hbm, scal_hbm, adj_hbm, hp_hbm, rs_hbm,
             src_idx, dst_idx, ss_buf, sd_buf, h_buf, scat_idx, e_buf,
             hp_sh, rs_sh, sem_g, sem_i):
        c = lax.axis_index("c")
        s = lax.axis_index("s")
        iota = lax.iota(jnp.int32, L)
        zeros = jnp.zeros((L,), jnp.float32)

        def edge_base(g):
            return c * (E // NC) + (g * NS + s) * CHUNK

        def start_idx_load(g, b):
            base = edge_base(g)
            pltpu.async_copy(adj_hbm.at[0, pl.ds(base, CHUNK)], src_idx[b],
                             sem_i[b])
            pltpu.async_copy(adj_hbm.at[1, pl.ds(base, CHUNK)], dst_idx[b],
                             sem_i[b])

        def wait_idx_load(g, b):
            base = edge_base(g)
            pltpu.make_async_copy(adj_hbm.at[0, pl.ds(base, CHUNK)], src_idx[b],
                                  sem_i[b]).wait()
            pltpu.make_async_copy(adj_hbm.at[1, pl.ds(base, CHUNK)], dst_idx[b],
                                  sem_i[b]).wait()

        def start_gathers(b):
            pltpu.async_copy(scal_hbm.at[src_idx[b]], ss_buf[b], sem_g[b])
            pltpu.async_copy(scal_hbm.at[dst_idx[b]], sd_buf[b], sem_g[b])
            pltpu.async_copy(h_hbm.at[dst_idx[b]], h_buf[b], sem_g[b])

        def wait_gathers(b):
            pltpu.make_async_copy(scal_hbm.at[src_idx[b]], ss_buf[b],
                                  sem_g[b]).wait()
            pltpu.make_async_copy(scal_hbm.at[dst_idx[b]], sd_buf[b],
                                  sem_g[b]).wait()
            pltpu.make_async_copy(h_hbm.at[dst_idx[b]], h_buf[b],
                                  sem_g[b]).wait()

        def compute(b):
            for i in range(n_heads):
                for t in range(groups):
                    r = t * L + iota
                    zs = plsc.load_gather(ss_buf[b],
                                          [r, jnp.full((L,), i, jnp.int32)])
                    zd = plsc.load_gather(sd_buf[b],
                                          [r, jnp.full((L,), n_heads + i,
                                                       jnp.int32)])
                    z = zs + zd
                    e = jnp.exp(-jnp.maximum(z, ALPHA * z))
                    plsc.store_scatter(e_buf, [r, jnp.full((L,), i, jnp.int32)], e)

            def scale_row(j, _):
                jv = jnp.full((L,), j, jnp.int32)
                for i in range(n_heads):
                    es = plsc.load_gather(e_buf, [jv, jnp.full((L,), i, jnp.int32)])
                    for k in range(NHID // L):
                        col = i * NHID + k * L
                        h_buf[b][j, pl.ds(col, L)] = h_buf[b][j, pl.ds(col, L)] * es
                return 0
            lax.fori_loop(0, CHUNK, scale_row, 0)

        def save_scat_idx(b):
            for t in range(CHUNK // L):
                scat_idx[pl.ds(t * L, L)] = src_idx[b][pl.ds(t * L, L)]

        def scatter(b):
            pltpu.sync_copy(h_buf[b], hp_sh.at[scat_idx], add=True)
            pltpu.sync_copy(e_buf, rs_sh.at[scat_idx], add=True)

        def zero_h(t, _):
            row = t // sub
            col = (t % sub) * L
            plsc.store_scatter(h_buf[0], [jnp.full((L,), row, jnp.int32),
                                          col + iota], zeros)
            return 0
        lax.fori_loop(0, CHUNK * sub, zero_h, 0)
        for t in range(CHUNK * 8 // L):
            f = t * L + iota
            plsc.store_scatter(e_buf, [f >> 3, f & 7], zeros)

        base_row = s * ROWS_PER_TILE
        n_full = ROWS_PER_TILE // CHUNK
        rem = ROWS_PER_TILE - n_full * CHUNK
        for k in range(n_full):
            pltpu.sync_copy(h_buf[0], hp_sh.at[pl.ds(base_row + k * CHUNK, CHUNK)])
            pltpu.sync_copy(e_buf, rs_sh.at[pl.ds(base_row + k * CHUNK, CHUNK)])
        pltpu.sync_copy(h_buf[0].at[pl.ds(0, rem)],
                        hp_sh.at[pl.ds(base_row + n_full * CHUNK, rem)])
        pltpu.sync_copy(e_buf.at[pl.ds(0, rem)],
                        rs_sh.at[pl.ds(base_row + n_full * CHUNK, rem)])
        plsc.subcore_barrier()

        pltpu.sync_copy(adj_hbm.at[0, pl.ds(edge_base(0), CHUNK)], src_idx[0])
        pltpu.sync_copy(adj_hbm.at[1, pl.ds(edge_base(0), CHUNK)], dst_idx[0])
        start_gathers(0)
        start_idx_load(1, 1)

        def stage(g, b):
            wait_idx_load(g + 1, 1 - b)
            start_gathers(1 - b)
            wait_gathers(b)
            save_scat_idx(b)
            if b == 0:
                start_idx_load(g + 2, b)
            else:
                @pl.when(g + 2 <= n_chunks - 1)
                def _():
                    start_idx_load(g + 2, b)
            compute(b)
            scatter(b)

        def pair_body(t, _):
            stage(2 * t, 0)
            stage(2 * t + 1, 1)
            return 0
        lax.fori_loop(0, (n_chunks - 1) // 2, pair_body, 0)

        wait_gathers(0)
        save_scat_idx(0)
        compute(0)
        scatter(0)
        plsc.subcore_barrier()

        pltpu.sync_copy(hp_sh.at[pl.ds(base_row, ROWS_PER_TILE)],
                        hp_hbm.at[c, pl.ds(base_row, ROWS_PER_TILE)])
        pltpu.sync_copy(rs_sh.at[pl.ds(base_row, ROWS_PER_TILE)],
                        rs_hbm.at[c, pl.ds(base_row, ROWS_PER_TILE)])

    return pl.kernel(
        body,
        out_type=(
            jax.ShapeDtypeStruct((NC, N, d_feat), jnp.float32),
            jax.ShapeDtypeStruct((NC, N, 8), jnp.float32),
        ),
        mesh=plsc.VectorSubcoreMesh(core_axis_name="c", subcore_axis_name="s"),
        compiler_params=pltpu.CompilerParams(use_tc_tiling_on_sc=False,
                                             needs_layout_passes=False),
        scratch_types=[
            (pltpu.VMEM((CHUNK,), jnp.int32),) * 2,
            (pltpu.VMEM((CHUNK,), jnp.int32),) * 2,
            (pltpu.VMEM((CHUNK, 8), jnp.float32),) * 2,
            (pltpu.VMEM((CHUNK, 8), jnp.float32),) * 2,
            (pltpu.VMEM((CHUNK, d_feat), jnp.float32),) * 2,
            pltpu.VMEM((CHUNK,), jnp.int32),
            pltpu.VMEM((CHUNK, 8), jnp.float32),
            pltpu.VMEM_SHARED((N, d_feat), jnp.float32),
            pltpu.VMEM_SHARED((N, 8), jnp.float32),
            (pltpu.SemaphoreType.DMA,) * 2,
            (pltpu.SemaphoreType.DMA,) * 2,
        ],
    )


@jax.jit
def kernel(x, adj, W0, a0, W1, a1, W2, a2, W_out, a_out):
    din = NHEADS * NHID
    wcat = jnp.concatenate([W0, W1, W2], axis=1)
    acat_a = jnp.zeros((din, 8), jnp.float32)
    for i, a in enumerate((a0, a1)):
        acat_a = acat_a.at[i * NHID:(i + 1) * NHID, i].set(a[0, :NHID])
        acat_a = acat_a.at[i * NHID:(i + 1) * NHID, 2 + i].set(a[0, NHID:])
    acat_b = jnp.zeros((din, 8), jnp.float32)
    acat_b = acat_b.at[2 * NHID:, 0].set(a2[0, :NHID])
    acat_b = acat_b.at[2 * NHID:, 1].set(a2[0, NHID:])
    aout = jnp.zeros((NCLASS, 8), jnp.float32)
    aout = aout.at[:, 0].set(a_out[0, :NCLASS])
    aout = aout.at[:, 1].set(a_out[0, NCLASS:])

    h01, h2h, scal_a, scal_b = _tc1(x, wcat, acat_a, acat_b)
    hpa, rsa = _sc_edge_kernel(2 * NHID, 2)(h01, scal_a, adj)
    hpb, rsb = _sc_edge_kernel(NHID, 1)(h2h, scal_b, adj)
    h2, scal2 = _tc2(hpa, rsa, hpb, rsb, W_out, aout)
    hp2, rs2 = _sc_edge_kernel(NCLASS, 1)(h2, scal2, adj)
    return _tc3(hp2, rs2)


# speedup vs baseline: 16.3162x; 1.2402x over previous
import jax
import jax.numpy as jnp
from jax import lax
from jax.experimental import pallas as pl
from jax.experimental.pallas import tpu as pltpu
from jax.experimental.pallas import tpu_sc as plsc

N = 10000
E = 320000
NFEAT = 128
NHID = 64
NCLASS = 64
NHEADS = 3
ALPHA = 0.2

NC = 2
NS = 16
L = 16
CHUNK = 80
CHUNKS_PER_TILE = (E // NC) // (NS * CHUNK)
ROWS_PER_TILE = N // NS

_ELU = lambda v: jnp.where(v > 0, v, jnp.exp(v) - 1.0)


def _tc1_body(x_ref, wcat_ref, acat_a_ref, acat_b_ref,
              ha_ref, hb_ref, scal_a_ref, scal_b_ref):
    h = jnp.dot(x_ref[...], wcat_ref[...], preferred_element_type=jnp.float32)
    scal_a = jnp.dot(h, acat_a_ref[...], preferred_element_type=jnp.float32)
    scal_b = jnp.dot(h, acat_b_ref[...], preferred_element_type=jnp.float32)
    ha_ref[...] = jnp.concatenate([h[:, :2 * NHID], scal_a], axis=1)
    hb_ref[...] = jnp.concatenate([h[:, 2 * NHID:], scal_b], axis=1)
    scal_a_ref[...] = scal_a
    scal_b_ref[...] = scal_b


def _tc1(x, wcat, acat_a, acat_b):
    blk = 1000
    return pl.pallas_call(
        _tc1_body,
        grid=(N // blk,),
        in_specs=[
            pl.BlockSpec((blk, NFEAT), lambda i: (i, 0)),
            pl.BlockSpec((NFEAT, NHEADS * NHID), lambda i: (0, 0)),
            pl.BlockSpec((NHEADS * NHID, 8), lambda i: (0, 0)),
            pl.BlockSpec((NHEADS * NHID, 8), lambda i: (0, 0)),
        ],
        out_specs=[
            pl.BlockSpec((blk, 2 * NHID + 8), lambda i: (i, 0)),
            pl.BlockSpec((blk, NHID + 8), lambda i: (i, 0)),
            pl.BlockSpec((blk, 8), lambda i: (i, 0)),
            pl.BlockSpec((blk, 8), lambda i: (i, 0)),
        ],
        out_shape=[
            jax.ShapeDtypeStruct((N, 2 * NHID + 8), jnp.float32),
            jax.ShapeDtypeStruct((N, NHID + 8), jnp.float32),
            jax.ShapeDtypeStruct((N, 8), jnp.float32),
            jax.ShapeDtypeStruct((N, 8), jnp.float32),
        ],
    )(x, wcat, acat_a, acat_b)


def _tc2_body(hpa_ref, hpb_ref, wout_ref, aout_ref, ha_ref, scal2_ref):
    hsa = hpa_ref[0] + hpa_ref[1]
    hsb = hpb_ref[0] + hpb_ref[1]
    parts = []
    for i in range(2):
        hi = hsa[:, i * NHID:(i + 1) * NHID] / (hsa[:, 2 * NHID + i:2 * NHID + i + 1]
                                                + 1e-16)
        parts.append(_ELU(hi))
    parts.append(_ELU(hsb[:, :NHID] / (hsb[:, NHID:NHID + 1] + 1e-16)))
    hcat = jnp.concatenate(parts, axis=1)
    h2 = jnp.dot(hcat, wout_ref[...], preferred_element_type=jnp.float32)
    scal2 = jnp.dot(h2, aout_ref[...], preferred_element_type=jnp.float32)
    ha_ref[...] = jnp.concatenate([h2, scal2], axis=1)
    scal2_ref[...] = scal2


def _tc2(hpa, hpb, wout, aout):
    blk = 1000
    return pl.pallas_call(
        _tc2_body,
        grid=(N // blk,),
        in_specs=[
            pl.BlockSpec((2, blk, 2 * NHID + 8), lambda i: (0, i, 0)),
            pl.BlockSpec((2, blk, NHID + 8), lambda i: (0, i, 0)),
            pl.BlockSpec((NHEADS * NHID, NCLASS), lambda i: (0, 0)),
            pl.BlockSpec((NCLASS, 8), lambda i: (0, 0)),
        ],
        out_specs=[
            pl.BlockSpec((blk, NCLASS + 8), lambda i: (i, 0)),
            pl.BlockSpec((blk, 8), lambda i: (i, 0)),
        ],
        out_shape=[
            jax.ShapeDtypeStruct((N, NCLASS + 8), jnp.float32),
            jax.ShapeDtypeStruct((N, 8), jnp.float32),
        ],
    )(hpa, hpb, wout, aout)


def _tc3_body(hp_ref, out_ref):
    hsum = hp_ref[0] + hp_ref[1]
    out_ref[...] = _ELU(hsum[:, :NCLASS] / (hsum[:, NCLASS:NCLASS + 1] + 1e-16))


def _tc3(hp):
    blk = 1000
    return pl.pallas_call(
        _tc3_body,
        grid=(N // blk,),
        in_specs=[pl.BlockSpec((2, blk, NCLASS + 8), lambda i: (0, i, 0))],
        out_specs=pl.BlockSpec((blk, NCLASS), lambda i: (i, 0)),
        out_shape=jax.ShapeDtypeStruct((N, NCLASS), jnp.float32),
    )(hp)


def _sc_edge_kernel(d_feat, n_heads):
    d_aug = d_feat + 8
    groups = CHUNK // L
    n_chunks = CHUNKS_PER_TILE

    def body(h_hbm, scal_hbm, adj_hbm, hp_hbm,
             src_idx, dst_idx, ss_buf, h_buf, scat_idx, hp_sh, sem_g, sem_i):
        c = lax.axis_index("c")
        s = lax.axis_index("s")
        iota = lax.iota(jnp.int32, L)
        zeros = jnp.zeros((L,), jnp.float32)

        def edge_base(g):
            return c * (E // NC) + (g * NS + s) * CHUNK

        def start_idx_load(g, b):
            base = edge_base(g)
            pltpu.async_copy(adj_hbm.at[0, pl.ds(base, CHUNK)], src_idx[b],
                             sem_i[b])
            pltpu.async_copy(adj_hbm.at[1, pl.ds(base, CHUNK)], dst_idx[b],
                             sem_i[b])

        def wait_idx_load(g, b):
            base = edge_base(g)
            pltpu.make_async_copy(adj_hbm.at[0, pl.ds(base, CHUNK)], src_idx[b],
                                  sem_i[b]).wait()
            pltpu.make_async_copy(adj_hbm.at[1, pl.ds(base, CHUNK)], dst_idx[b],
                                  sem_i[b]).wait()

        def start_gathers(b):
            pltpu.async_copy(scal_hbm.at[src_idx[b]], ss_buf[b], sem_g[b])
            pltpu.async_copy(h_hbm.at[dst_idx[b]], h_buf[b], sem_g[b])

        def wait_gathers(b):
            pltpu.make_async_copy(scal_hbm.at[src_idx[b]], ss_buf[b],
                                  sem_g[b]).wait()
            pltpu.make_async_copy(h_hbm.at[dst_idx[b]], h_buf[b],
                                  sem_g[b]).wait()

        def compute(b):
            for i in range(n_heads):
                for t in range(groups):
                    r = t * L + iota
                    zs = plsc.load_gather(ss_buf[b],
                                          [r, jnp.full((L,), i, jnp.int32)])
                    zd = plsc.load_gather(h_buf[b],
                                          [r, jnp.full((L,), d_feat + n_heads + i,
                                                       jnp.int32)])
                    z = zs + zd
                    e = jnp.exp(-jnp.maximum(z, ALPHA * z))
                    plsc.store_scatter(h_buf[b],
                                       [r, jnp.full((L,), d_feat + i, jnp.int32)],
                                       e)

            @plsc.parallel_loop(0, CHUNK, 1, unroll=2)
            def _(j):
                jv = jnp.full((L,), j, jnp.int32)
                for i in range(n_heads):
                    es = plsc.load_gather(h_buf[b],
                                          [jv, jnp.full((L,), d_feat + i,
                                                        jnp.int32)])
                    for k in range(NHID // L):
                        col = i * NHID + k * L
                        h_buf[b][j, pl.ds(col, L)] = h_buf[b][j, pl.ds(col, L)] * es

        def save_scat_idx(b):
            for t in range(CHUNK // L):
                scat_idx[pl.ds(t * L, L)] = src_idx[b][pl.ds(t * L, L)]

        def scatter(b):
            pltpu.sync_copy(h_buf[b], hp_sh.at[scat_idx], add=True)

        def zero_h(t, _):
            f = t * L + iota
            plsc.store_scatter(h_buf[0], [f // d_aug, f % d_aug], zeros)
            return 0
        lax.fori_loop(0, CHUNK * d_aug // L, zero_h, 0)

        base_row = s * ROWS_PER_TILE
        n_full = ROWS_PER_TILE // CHUNK
        rem = ROWS_PER_TILE - n_full * CHUNK
        for k in range(n_full):
            pltpu.sync_copy(h_buf[0], hp_sh.at[pl.ds(base_row + k * CHUNK, CHUNK)])
        pltpu.sync_copy(h_buf[0].at[pl.ds(0, rem)],
                        hp_sh.at[pl.ds(base_row + n_full * CHUNK, rem)])
        plsc.subcore_barrier()

        pltpu.sync_copy(adj_hbm.at[0, pl.ds(edge_base(0), CHUNK)], src_idx[0])
        pltpu.sync_copy(adj_hbm.at[1, pl.ds(edge_base(0), CHUNK)], dst_idx[0])
        start_gathers(0)
        start_idx_load(1, 1)

        def stage(g, b):
            wait_idx_load(g + 1, 1 - b)
            start_gathers(1 - b)
            wait_gathers(b)
            save_scat_idx(b)
            if b == 0:
                start_idx_load(g + 2, b)
            else:
                @pl.when(g + 2 <= n_chunks - 1)
                def _():
                    start_idx_load(g + 2, b)
            compute(b)
            scatter(b)

        def pair_body(t, _):
            stage(2 * t, 0)
            stage(2 * t + 1, 1)
            return 0
        lax.fori_loop(0, (n_chunks - 1) // 2, pair_body, 0)

        wait_gathers(0)
        save_scat_idx(0)
        compute(0)
        scatter(0)
        plsc.subcore_barrier()

        pltpu.sync_copy(hp_sh.at[pl.ds(base_row, ROWS_PER_TILE)],
                        hp_hbm.at[c, pl.ds(base_row, ROWS_PER_TILE)])

    return pl.kernel(
        body,
        out_type=jax.ShapeDtypeStruct((NC, N, d_aug), jnp.float32),
        mesh=plsc.VectorSubcoreMesh(core_axis_name="c", subcore_axis_name="s"),
        compiler_params=pltpu.CompilerParams(use_tc_tiling_on_sc=False,
                                             needs_layout_passes=False),
        scratch_types=[
            (pltpu.VMEM((CHUNK,), jnp.int32),) * 2,
            (pltpu.VMEM((CHUNK,), jnp.int32),) * 2,
            (pltpu.VMEM((CHUNK, 8), jnp.float32),) * 2,
            (pltpu.VMEM((CHUNK, d_aug), jnp.float32),) * 2,
            pltpu.VMEM((CHUNK,), jnp.int32),
            pltpu.VMEM_SHARED((N, d_aug), jnp.float32),
            (pltpu.SemaphoreType.DMA,) * 2,
            (pltpu.SemaphoreType.DMA,) * 2,
        ],
    )


@jax.jit
def kernel(x, adj, W0, a0, W1, a1, W2, a2, W_out, a_out):
    din = NHEADS * NHID
    wcat = jnp.concatenate([W0, W1, W2], axis=1)
    acat_a = jnp.zeros((din, 8), jnp.float32)
    for i, a in enumerate((a0, a1)):
        acat_a = acat_a.at[i * NHID:(i + 1) * NHID, i].set(a[0, :NHID])
        acat_a = acat_a.at[i * NHID:(i + 1) * NHID, 2 + i].set(a[0, NHID:])
    acat_b = jnp.zeros((din, 8), jnp.float32)
    acat_b = acat_b.at[2 * NHID:, 0].set(a2[0, :NHID])
    acat_b = acat_b.at[2 * NHID:, 1].set(a2[0, NHID:])
    aout = jnp.zeros((NCLASS, 8), jnp.float32)
    aout = aout.at[:, 0].set(a_out[0, :NCLASS])
    aout = aout.at[:, 1].set(a_out[0, NCLASS:])

    ha, hb, scal_a, scal_b = _tc1(x, wcat, acat_a, acat_b)
    hpa = _sc_edge_kernel(2 * NHID, 2)(ha, scal_a, adj)
    hpb = _sc_edge_kernel(NHID, 1)(hb, scal_b, adj)
    ha2, scal2 = _tc2(hpa, hpb, W_out, aout)
    hp2 = _sc_edge_kernel(NCLASS, 1)(ha2, scal2, adj)
    return _tc3(hp2)
